# baseline (speedup 1.0000x reference)
import functools

import jax
import jax.numpy as jnp
from jax import lax
from jax.experimental import pallas as pl
from jax.experimental.pallas import tpu as pltpu
from jax.experimental.pallas import tpu_sc as plsc

_NC = 2
_NS = 16
_NW = _NC * _NS
_L = 16


def _lookup_t(tab128, xw_t, seq, dim):
    _, b = xw_t.shape
    bpw = b // _NW
    ngrp = seq // 2

    mesh = plsc.VectorSubcoreMesh(core_axis_name="c", subcore_axis_name="s")

    @functools.partial(
        pl.kernel,
        mesh=mesh,
        compiler_params=pltpu.CompilerParams(
            use_tc_tiling_on_sc=False, needs_layout_passes=False
        ),
        out_type=jax.ShapeDtypeStruct((seq, dim, b), jnp.float32),
        scratch_types=[
            pltpu.VMEM((seq, bpw), jnp.int32),
            pltpu.VMEM((2, bpw, 128), jnp.float32),
            pltpu.VMEM((2, dim, bpw), jnp.float32),
            pltpu.SemaphoreType.DMA,
            pltpu.SemaphoreType.DMA,
            pltpu.SemaphoreType.DMA,
            pltpu.SemaphoreType.DMA,
        ],
    )
    def k(tab_hbm, xw_hbm, out_hbm, idx_v, gbuf, tbuf, g0, g1, o0, o1):
        wid = lax.axis_index("s") * _NC + lax.axis_index("c")
        col0 = wid * bpw
        gsems = (g0, g1)
        osems = (o0, o1)
        pltpu.sync_copy(xw_hbm.at[:, pl.ds(col0, bpw)], idx_v)

        biota = [
            (jnp.arange(_L, dtype=jnp.int32) + g * _L) for g in range(bpw // _L)
        ]

        def gather(s, buf):
            return pltpu.make_async_copy(
                tab_hbm.at[idx_v.at[s]], gbuf.at[buf], gsems[buf]
            )

        def write(s, buf):
            return pltpu.make_async_copy(
                tbuf.at[buf], out_hbm.at[s, :, pl.ds(col0, bpw)], osems[buf]
            )

        gather(0, 0).start()
        gather(1, 1).start()

        def body(g, _):
            for buf in range(2):
                s = g * 2 + buf
                gather(s, buf).wait()

                @pl.when(g >= 1)
                def _():
                    write(s - 2, buf).wait()

                for d in range(dim):
                    cold = jnp.full((_L,), d, jnp.int32)
                    for q in range(bpw // _L):
                        tbuf[buf, d, pl.ds(q * _L, _L)] = plsc.load_gather(
                            gbuf.at[buf], [biota[q], cold]
                        )
                write(s, buf).start()

                @pl.when(g < ngrp - 1)
                def _():
                    gather(s + 2, buf).start()

            return 0

        lax.fori_loop(0, ngrp, body, 0)
        write(seq - 2, 0).wait()
        write(seq - 1, 1).wait()

    return k(tab128, xw_t)


def kernel(X_word, X_char, word_embed):
    batch, seq = X_word.shape
    v, dim = word_embed.shape
    tab128 = jnp.pad(word_embed, ((0, 0), (0, 128 - dim)))
    xw_t = jnp.transpose(X_word.astype(jnp.int32))
    out_t = _lookup_t(tab128, xw_t, seq, dim)
    return jnp.transpose(out_t, (2, 0, 1))


# speedup vs baseline: 1.5208x; 1.5208x over previous
import functools

import jax
import jax.numpy as jnp
from jax import lax
from jax.experimental import pallas as pl
from jax.experimental.pallas import tpu as pltpu
from jax.experimental.pallas import tpu_sc as plsc

_NC = 2
_NS = 16
_NW = _NC * _NS
_L = 16


def _lookup_t(tab128, xw_t, seq, dim):
    _, b = xw_t.shape
    bpw = b // _NW
    ngrp = seq // 2

    mesh = plsc.VectorSubcoreMesh(core_axis_name="c", subcore_axis_name="s")

    @functools.partial(
        pl.kernel,
        mesh=mesh,
        compiler_params=pltpu.CompilerParams(
            use_tc_tiling_on_sc=False, needs_layout_passes=False
        ),
        out_type=jax.ShapeDtypeStruct((seq, dim, b), jnp.float32),
        scratch_types=[
            pltpu.VMEM((seq, bpw), jnp.int32),
            pltpu.VMEM((2, bpw, 128), jnp.float32),
            pltpu.VMEM((2, dim, bpw), jnp.float32),
            pltpu.SemaphoreType.DMA,
            pltpu.SemaphoreType.DMA,
            pltpu.SemaphoreType.DMA,
            pltpu.SemaphoreType.DMA,
        ],
    )
    def k(tab_hbm, xw_hbm, out_hbm, idx_v, gbuf, tbuf, g0, g1, o0, o1):
        wid = lax.axis_index("s") * _NC + lax.axis_index("c")
        col0 = wid * bpw
        gsems = (g0, g1)
        osems = (o0, o1)
        pltpu.sync_copy(xw_hbm.at[:, pl.ds(col0, bpw)], idx_v)

        biota = [
            (jnp.arange(_L, dtype=jnp.int32) + g * _L) for g in range(bpw // _L)
        ]

        def gather(s, buf):
            return pltpu.make_async_copy(
                tab_hbm.at[idx_v.at[s]], gbuf.at[buf], gsems[buf]
            )

        def write(s, buf):
            return pltpu.make_async_copy(
                tbuf.at[buf], out_hbm.at[s, :, pl.ds(col0, bpw)], osems[buf]
            )

        gather(0, 0).start()
        gather(1, 1).start()

        def body(g, _):
            for buf in range(2):
                s = g * 2 + buf
                gather(s, buf).wait()

                @pl.when(g >= 1)
                def _():
                    write(s - 2, buf).wait()

                @plsc.parallel_loop(0, dim, unroll=8)
                def _(d):
                    cold = jnp.zeros((_L,), jnp.int32) + d
                    for q in range(bpw // _L):
                        tbuf[buf, d, pl.ds(q * _L, _L)] = plsc.load_gather(
                            gbuf.at[buf], [biota[q], cold]
                        )

                write(s, buf).start()

                @pl.when(g < ngrp - 1)
                def _():
                    gather(s + 2, buf).start()

            return 0

        lax.fori_loop(0, ngrp, body, 0)
        write(seq - 2, 0).wait()
        write(seq - 1, 1).wait()

    return k(tab128, xw_t)


def kernel(X_word, X_char, word_embed):
    batch, seq = X_word.shape
    v, dim = word_embed.shape
    tab128 = jnp.pad(word_embed, ((0, 0), (0, 128 - dim)))
    xw_t = jnp.transpose(X_word.astype(jnp.int32))
    out_t = _lookup_t(tab128, xw_t, seq, dim)
    return jnp.transpose(out_t, (2, 0, 1))


# speedup vs baseline: 1.5244x; 1.0023x over previous
import functools

import jax
import jax.numpy as jnp
from jax import lax
from jax.experimental import pallas as pl
from jax.experimental.pallas import tpu as pltpu
from jax.experimental.pallas import tpu_sc as plsc

_NC = 2
_NS = 16
_NW = _NC * _NS
_L = 16


def _lookup_t(tab128, xw_t, seq, dim):
    _, b = xw_t.shape
    bpw = b // _NW
    nbuf = 4
    ngrp = seq // nbuf

    mesh = plsc.VectorSubcoreMesh(core_axis_name="c", subcore_axis_name="s")

    @functools.partial(
        pl.kernel,
        mesh=mesh,
        compiler_params=pltpu.CompilerParams(
            use_tc_tiling_on_sc=False, needs_layout_passes=False
        ),
        out_type=jax.ShapeDtypeStruct((seq, dim, b), jnp.float32),
        scratch_types=[
            pltpu.VMEM((seq, bpw), jnp.int32),
            pltpu.VMEM((4, bpw, 128), jnp.float32),
            pltpu.VMEM((4, dim, bpw), jnp.float32),
            pltpu.SemaphoreType.DMA,
            pltpu.SemaphoreType.DMA,
            pltpu.SemaphoreType.DMA,
            pltpu.SemaphoreType.DMA,
            pltpu.SemaphoreType.DMA,
            pltpu.SemaphoreType.DMA,
            pltpu.SemaphoreType.DMA,
            pltpu.SemaphoreType.DMA,
        ],
    )
    def k(tab_hbm, xw_hbm, out_hbm, idx_v, gbuf, tbuf,
          g0, g1, g2, g3, o0, o1, o2, o3):
        wid = lax.axis_index("s") * _NC + lax.axis_index("c")
        col0 = wid * bpw
        gsems = (g0, g1, g2, g3)
        osems = (o0, o1, o2, o3)
        pltpu.sync_copy(xw_hbm.at[:, pl.ds(col0, bpw)], idx_v)

        biota = [
            (jnp.arange(_L, dtype=jnp.int32) + g * _L) for g in range(bpw // _L)
        ]

        def gather(s, buf):
            return pltpu.make_async_copy(
                tab_hbm.at[idx_v.at[s]], gbuf.at[buf], gsems[buf]
            )

        def write(s, buf):
            return pltpu.make_async_copy(
                tbuf.at[buf], out_hbm.at[s, :, pl.ds(col0, bpw)], osems[buf]
            )

        for p in range(nbuf):
            gather(p, p).start()

        def body(g, _):
            for buf in range(nbuf):
                s = g * nbuf + buf
                gather(s, buf).wait()

                @pl.when(g >= 1)
                def _():
                    write(s - nbuf, buf).wait()

                @plsc.parallel_loop(0, dim, unroll=8)
                def _(d):
                    cold = jnp.zeros((_L,), jnp.int32) + d
                    for q in range(bpw // _L):
                        tbuf[buf, d, pl.ds(q * _L, _L)] = plsc.load_gather(
                            gbuf.at[buf], [biota[q], cold]
                        )

                write(s, buf).start()

                @pl.when(g < ngrp - 1)
                def _():
                    gather(s + nbuf, buf).start()

            return 0

        lax.fori_loop(0, ngrp, body, 0)
        for p in range(nbuf):
            write(seq - nbuf + p, p).wait()

    return k(tab128, xw_t)


def kernel(X_word, X_char, word_embed):
    batch, seq = X_word.shape
    v, dim = word_embed.shape
    tab128 = jnp.pad(word_embed, ((0, 0), (0, 128 - dim)))
    xw_t = jnp.transpose(X_word.astype(jnp.int32))
    out_t = _lookup_t(tab128, xw_t, seq, dim)
    return jnp.transpose(out_t, (2, 0, 1))


# speedup vs baseline: 1.5560x; 1.0207x over previous
import functools

import jax
import jax.numpy as jnp
from jax import lax
from jax.experimental import pallas as pl
from jax.experimental.pallas import tpu as pltpu
from jax.experimental.pallas import tpu_sc as plsc

_NC = 2
_NS = 16
_NW = _NC * _NS
_L = 16


def _lookup_t(tab128, xw_t, seq, dim):
    _, b = xw_t.shape
    bpw = b // _NW
    nbuf = 4
    ngrp = seq // nbuf

    mesh = plsc.VectorSubcoreMesh(core_axis_name="c", subcore_axis_name="s")

    @functools.partial(
        pl.kernel,
        mesh=mesh,
        compiler_params=pltpu.CompilerParams(
            use_tc_tiling_on_sc=False, needs_layout_passes=False
        ),
        out_type=jax.ShapeDtypeStruct((seq * dim * (b // 128), 128), jnp.float32),
        scratch_types=[
            pltpu.VMEM((seq, bpw), jnp.int32),
            pltpu.VMEM((4, bpw, 128), jnp.float32),
            pltpu.VMEM((4, dim, bpw), jnp.float32),
            pltpu.VMEM((4, dim), jnp.int32),
            pltpu.SemaphoreType.DMA,
            pltpu.SemaphoreType.DMA,
            pltpu.SemaphoreType.DMA,
            pltpu.SemaphoreType.DMA,
            pltpu.SemaphoreType.DMA,
            pltpu.SemaphoreType.DMA,
            pltpu.SemaphoreType.DMA,
            pltpu.SemaphoreType.DMA,
        ],
    )
    def k(tab_hbm, xw_hbm, out_hbm, idx_v, gbuf, tbuf, oidx_v,
          g0, g1, g2, g3, o0, o1, o2, o3):
        wid = lax.axis_index("s") * _NC + lax.axis_index("c")
        col0 = wid * bpw
        gsems = (g0, g1, g2, g3)
        osems = (o0, o1, o2, o3)
        pltpu.sync_copy(xw_hbm.at[:, pl.ds(col0, bpw)], idx_v)

        biota = [
            (jnp.arange(_L, dtype=jnp.int32) + g * _L) for g in range(bpw // _L)
        ]

        def gather(s, buf):
            return pltpu.make_async_copy(
                tab_hbm.at[idx_v.at[s]], gbuf.at[buf], gsems[buf]
            )

        nblk = b // 128
        diota = [
            (jnp.arange(_L, dtype=jnp.int32) + g * _L) * nblk
            for g in range(dim // _L)
        ]

        def write(s, buf):
            return pltpu.make_async_copy(
                tbuf.at[buf], out_hbm.at[oidx_v.at[buf]], osems[buf]
            )

        def set_oidx(s, buf):
            base = s * dim * nblk + wid
            for g in range(dim // _L):
                oidx_v[buf, pl.ds(g * _L, _L)] = diota[g] + base

        for p in range(nbuf):
            set_oidx(p, p)
            gather(p, p).start()

        def body(g, _):
            for buf in range(nbuf):
                s = g * nbuf + buf
                gather(s, buf).wait()

                @pl.when(g >= 1)
                def _():
                    write(s - nbuf, buf).wait()

                @plsc.parallel_loop(0, dim, unroll=4)
                def _(d):
                    cold = jnp.zeros((_L,), jnp.int32) + d
                    for q in range(bpw // _L):
                        tbuf[buf, d, pl.ds(q * _L, _L)] = plsc.load_gather(
                            gbuf.at[buf], [biota[q], cold]
                        )

                @pl.when(g >= 1)
                def _():
                    set_oidx(s, buf)

                write(s, buf).start()

                @pl.when(g < ngrp - 1)
                def _():
                    gather(s + nbuf, buf).start()

            return 0

        lax.fori_loop(0, ngrp, body, 0)
        for p in range(nbuf):
            write(seq - nbuf + p, p).wait()

    return k(tab128, xw_t)


def kernel(X_word, X_char, word_embed):
    batch, seq = X_word.shape
    v, dim = word_embed.shape
    tab128 = jnp.pad(word_embed, ((0, 0), (0, 128 - dim)))
    xw_t = jnp.transpose(X_word.astype(jnp.int32))
    out2 = _lookup_t(tab128, xw_t, seq, dim)
    out_t = out2.reshape(seq, dim, batch)
    return jnp.transpose(out_t, (2, 0, 1))


# speedup vs baseline: 1.7626x; 1.1328x over previous
import functools

import jax
import jax.numpy as jnp
from jax import lax
from jax.experimental import pallas as pl
from jax.experimental.pallas import tpu as pltpu
from jax.experimental.pallas import tpu_sc as plsc

_NC = 2
_NS = 16
_NW = _NC * _NS

_QB = 2


def _lookup(table, xw):
    b, s = xw.shape
    _, d = table.shape
    rpw = b // _NW
    n_bursts = rpw // _QB
    s0 = (s // 2 + 127) // 128 * 128
    s1 = s - s0

    mesh = plsc.VectorSubcoreMesh(core_axis_name="c", subcore_axis_name="s")

    @functools.partial(
        pl.kernel,
        mesh=mesh,
        compiler_params=pltpu.CompilerParams(use_tc_tiling_on_sc=False),
        out_type=jax.ShapeDtypeStruct((b, s, d), jnp.float32),
        scratch_types=[
            pltpu.VMEM((rpw, s), jnp.int32),
            pltpu.VMEM((2, _QB, s, d), jnp.float32),
            pltpu.SemaphoreType.DMA,
            pltpu.SemaphoreType.DMA,
            pltpu.SemaphoreType.DMA,
            pltpu.SemaphoreType.DMA,
        ],
    )
    def k(table_hbm, xw_hbm, out_hbm, idx_v, rows_v, g0, g1, o0, o1):
        wid = lax.axis_index("s") * _NC + lax.axis_index("c")
        row_base = wid * rpw
        gsems = (g0, g1)
        osems = (o0, o1)
        pltpu.sync_copy(xw_hbm.at[pl.ds(row_base, rpw)], idx_v)

        def fire(t, buf):
            for q in range(_QB):
                r = t * _QB + q
                pltpu.async_copy(
                    table_hbm.at[idx_v.at[r, pl.ds(0, s0)]],
                    rows_v.at[buf, q, pl.ds(0, s0)],
                    gsems[buf],
                )
                pltpu.async_copy(
                    table_hbm.at[idx_v.at[r, pl.ds(s0, s1)]],
                    rows_v.at[buf, q, pl.ds(s0, s1)],
                    gsems[buf],
                )

        fire(0, 0)
        fire(1, 1)

        def body(g, _):
            for buf in range(2):
                t = g * 2 + buf
                pltpu.make_async_copy(
                    out_hbm.at[pl.ds(row_base, _QB)], rows_v.at[buf], gsems[buf]
                ).wait()
                w = pltpu.async_copy(
                    rows_v.at[buf],
                    out_hbm.at[pl.ds(row_base + t * _QB, _QB)],
                    osems[buf],
                )
                w.wait()

                @pl.when(t + 2 < n_bursts)
                def _():
                    fire(t + 2, buf)

            return 0

        lax.fori_loop(0, n_bursts // 2, body, 0)

    return k(table, xw)


def kernel(X_word, X_char, word_embed):
    return _lookup(word_embed, X_word.astype(jnp.int32))
